# Initial kernel scaffold; baseline (speedup 1.0000x reference)
import jax
import jax.numpy as jnp
from jax.experimental import pallas as pl


def kernel(x, edge_index, W1, b1, W2, b2, W3, b3):
    raise NotImplementedError("write your pallas kernel here")


# speedup vs baseline: 14.1056x; 14.1056x over previous
import functools

import jax
import jax.numpy as jnp
from jax import lax
from jax.experimental import pallas as pl
from jax.experimental.pallas import tpu as pltpu
from jax.experimental.pallas import tpu_sc as plsc

N = 10000
E = 320000
D = 128

NC = 2
NS = 16
NW = NC * NS
EPW = E // NW
CH = 80
NCHUNK = EPW // CH
RPT = N // NS
ZR = 125

def _zero_vec16():
    return jnp.zeros((16,), jnp.float32)


@functools.cache
def _deg_kernel():
    return pl.kernel(
        _deg_body,
        out_type=jax.ShapeDtypeStruct((NW * N,), jnp.float32),
        mesh=plsc.VectorSubcoreMesh(core_axis_name="c", subcore_axis_name="s"),
        compiler_params=pltpu.CompilerParams(needs_layout_passes=False),
        scratch_types=[
            pltpu.VMEM((N,), jnp.float32),
            pltpu.VMEM((CH,), jnp.int32),
        ],
    )


def _deg_body(dst_hbm, out_hbm, hist, dbuf):
    c = lax.axis_index("c")
    s = lax.axis_index("s")
    wid = s * NC + c

    def zero_body(i, carry):
        hist[pl.ds(i * 16, 16)] = _zero_vec16()
        return carry

    lax.fori_loop(0, N // 16, zero_body, 0)

    ones = jnp.ones((16,), jnp.float32)

    def body(k, carry):
        base = wid * EPW + k * CH
        pltpu.sync_copy(dst_hbm.at[pl.ds(base, CH)], dbuf)
        for j in range(CH // 16):
            d16 = dbuf[pl.ds(j * 16, 16)]
            plsc.addupdate_scatter(hist, [d16], ones)
        return carry

    lax.fori_loop(0, NCHUNK, body, 0)
    pltpu.sync_copy(hist, out_hbm.at[pl.ds(wid * N, N)])


@functools.cache
def _agg_kernel():
    return pl.kernel(
        _agg_body,
        out_type=jax.ShapeDtypeStruct((NC * N, D), jnp.float32),
        mesh=plsc.VectorSubcoreMesh(core_axis_name="c", subcore_axis_name="s"),
        compiler_params=pltpu.CompilerParams(needs_layout_passes=False),
        scratch_types=[
            pltpu.VMEM_SHARED((N, D), jnp.float32),
            pltpu.VMEM((CH, D), jnp.float32),
            pltpu.VMEM((CH,), jnp.int32),
            pltpu.VMEM((CH,), jnp.int32),
            pltpu.SemaphoreType.DMA,
        ],
    )


NRCH = N // CH
RCPT = (NRCH + NS - 1) // NS


def _agg_body(hp_hbm, src_hbm, dst_hbm, out_hbm, acc, rows, sbuf, dbuf, sem):
    c = lax.axis_index("c")
    s = lax.axis_index("s")
    wid = s * NC + c

    def zrow(i, carry):
        for j in range(D // 16):
            rows[i, pl.ds(j * 16, 16)] = _zero_vec16()
        return carry

    lax.fori_loop(0, CH, zrow, 0)
    for r in range(RCPT):
        chunk = s + r * NS
        @pl.when(chunk < NRCH)
        def _():
            pltpu.sync_copy(rows, acc.at[pl.ds(chunk * CH, CH), :])
    plsc.subcore_barrier()

    def body(k, carry):
        base = wid * EPW + k * CH
        pltpu.sync_copy(src_hbm.at[pl.ds(base, CH)], sbuf)
        pltpu.sync_copy(dst_hbm.at[pl.ds(base, CH)], dbuf)
        pltpu.async_copy(hp_hbm.at[sbuf], rows, sem).wait()
        pltpu.sync_copy(rows, acc.at[dbuf], add=True)
        return carry

    lax.fori_loop(0, NCHUNK, body, 0)
    plsc.subcore_barrier()
    for r in range(RCPT):
        chunk = s + r * NS
        @pl.when(chunk < NRCH)
        def _():
            pltpu.sync_copy(
                acc.at[pl.ds(chunk * CH, CH), :],
                out_hbm.at[pl.ds(c * N + chunk * CH, CH), :],
            )


@functools.cache
def _agg1_kernel():
    return pl.kernel(
        _agg1_body,
        out_type=jax.ShapeDtypeStruct((NW * N,), jnp.float32),
        mesh=plsc.VectorSubcoreMesh(core_axis_name="c", subcore_axis_name="s"),
        compiler_params=pltpu.CompilerParams(needs_layout_passes=False),
        scratch_types=[
            pltpu.VMEM((N,), jnp.float32),
            pltpu.VMEM((N,), jnp.float32),
            pltpu.VMEM((CH,), jnp.int32),
            pltpu.VMEM((CH,), jnp.int32),
        ],
    )


def _agg1_body(z_hbm, src_hbm, dst_hbm, out_hbm, zp, acc, sbuf, dbuf):
    c = lax.axis_index("c")
    s = lax.axis_index("s")
    wid = s * NC + c

    pltpu.sync_copy(z_hbm, zp)

    def zero_body(i, carry):
        acc[pl.ds(i * 16, 16)] = _zero_vec16()
        return carry

    lax.fori_loop(0, N // 16, zero_body, 0)

    def body(k, carry):
        base = wid * EPW + k * CH
        pltpu.sync_copy(src_hbm.at[pl.ds(base, CH)], sbuf)
        pltpu.sync_copy(dst_hbm.at[pl.ds(base, CH)], dbuf)
        for j in range(CH // 16):
            s16 = sbuf[pl.ds(j * 16, 16)]
            d16 = dbuf[pl.ds(j * 16, 16)]
            vals = plsc.load_gather(zp, [s16])
            plsc.addupdate_scatter(acc, [d16], vals)
        return carry

    lax.fori_loop(0, NCHUNK, body, 0)
    pltpu.sync_copy(acc, out_hbm.at[pl.ds(wid * N, N)])


_RB = 2000


def _prep_body(dp_ref, o_ref):
    deg = jnp.sum(dp_ref[...], axis=0, keepdims=True) + 1.0
    o_ref[...] = lax.rsqrt(deg)


def _prep(degparts):
    return pl.pallas_call(
        _prep_body,
        out_shape=jax.ShapeDtypeStruct((1, N), jnp.float32),
    )(degparts)


def _mm_body(x_ref, w_ref, dinv_ref, o_ref):
    p = jnp.dot(x_ref[...], w_ref[...], preferred_element_type=jnp.float32)
    o_ref[...] = p * dinv_ref[...]


def _mm(x, W, dinv_col):
    kd = W.shape[0]
    od = W.shape[1]
    return pl.pallas_call(
        _mm_body,
        out_shape=jax.ShapeDtypeStruct((N, od), jnp.float32),
        grid=(N // _RB,),
        in_specs=[
            pl.BlockSpec((_RB, kd), lambda i: (i, 0)),
            pl.BlockSpec((kd, od), lambda i: (0, 0)),
            pl.BlockSpec((_RB, 1), lambda i: (i, 0)),
        ],
        out_specs=pl.BlockSpec((_RB, od), lambda i: (i, 0)),
    )(x, W, dinv_col)


def _comb_body(relu, p0_ref, p1_ref, pp_ref, dinv_ref, b_ref, o_ref):
    v = dinv_ref[...] * (p0_ref[...] + p1_ref[...] + pp_ref[...]) + b_ref[...]
    if relu:
        v = jnp.maximum(v, 0.0)
    o_ref[...] = v


def _comb(p0, p1, pp, dinv_col, bias_row, relu):
    return pl.pallas_call(
        functools.partial(_comb_body, relu),
        out_shape=jax.ShapeDtypeStruct((N, D), jnp.float32),
        grid=(N // _RB,),
        in_specs=[
            pl.BlockSpec((_RB, D), lambda i: (i, 0)),
            pl.BlockSpec((_RB, D), lambda i: (i, 0)),
            pl.BlockSpec((_RB, D), lambda i: (i, 0)),
            pl.BlockSpec((_RB, 1), lambda i: (i, 0)),
            pl.BlockSpec((1, D), lambda i: (0, 0)),
        ],
        out_specs=pl.BlockSpec((_RB, D), lambda i: (i, 0)),
    )(p0, p1, pp, dinv_col, bias_row)


def _comb3_body(parts_ref, z_ref, dinv_ref, b_ref, o_ref):
    agg = jnp.sum(parts_ref[...], axis=0, keepdims=True)
    o_ref[...] = dinv_ref[...] * (agg + z_ref[...]) + b_ref[0, 0]


def _comb3(parts3, z_row, dinv_row, b3):
    return pl.pallas_call(
        _comb3_body,
        out_shape=jax.ShapeDtypeStruct((1, N), jnp.float32),
    )(parts3, z_row, dinv_row, b3)


@jax.jit
def kernel(x, edge_index, W1, b1, W2, b2, W3, b3):
    src = edge_index[0].astype(jnp.int32)
    dst = edge_index[1].astype(jnp.int32)

    degparts = _deg_kernel()(dst).reshape(NW, N)
    dinv_row = _prep(degparts)
    dinv_col = dinv_row.reshape(N, 1)

    def layer128(xi, W, b, relu):
        pp = _mm(xi, W, dinv_col)
        parts = _agg_kernel()(pp, src, dst)
        return _comb(parts[:N], parts[N:], pp, dinv_col, b.reshape(1, D), relu)

    h = layer128(x, W1, b1, True)
    h = layer128(h, W2, b2, True)

    z = _mm(h, W3, dinv_col)
    parts3 = _agg1_kernel()(z.reshape(N), src, dst).reshape(NW, N)
    out_row = _comb3(parts3, z.reshape(1, N), dinv_row, b3.reshape(1, 1))
    return out_row.reshape(N, 1)


# speedup vs baseline: 32.0312x; 2.2708x over previous
import functools

import jax
import jax.numpy as jnp
from jax import lax
from jax.experimental import pallas as pl
from jax.experimental.pallas import tpu as pltpu
from jax.experimental.pallas import tpu_sc as plsc

N = 10000
E = 320000
D = 128

NC = 2
NS = 16
NW = NC * NS
EPW = E // NW
CH = 80
NCHUNK = EPW // CH
NB = 4

def _zero_vec16():
    return jnp.zeros((16,), jnp.float32)


@functools.cache
def _deg_kernel():
    return pl.kernel(
        _deg_body,
        out_type=jax.ShapeDtypeStruct((NW * N,), jnp.float32),
        mesh=plsc.VectorSubcoreMesh(core_axis_name="c", subcore_axis_name="s"),
        compiler_params=pltpu.CompilerParams(needs_layout_passes=False),
        scratch_types=[
            pltpu.VMEM((N,), jnp.float32),
            pltpu.VMEM((EPW,), jnp.int32),
        ],
    )


def _deg_body(dst_hbm, out_hbm, hist, dbuf):
    c = lax.axis_index("c")
    s = lax.axis_index("s")
    wid = s * NC + c

    pltpu.sync_copy(dst_hbm.at[pl.ds(wid * EPW, EPW)], dbuf)

    def zero_body(i, carry):
        hist[pl.ds(i * 16, 16)] = _zero_vec16()
        return carry

    lax.fori_loop(0, N // 16, zero_body, 0)

    ones = jnp.ones((16,), jnp.float32)

    def body(j, carry):
        d16 = dbuf[pl.ds(j * 16, 16)]
        plsc.addupdate_scatter(hist, [d16], ones)
        return carry

    lax.fori_loop(0, EPW // 16, body, 0)
    pltpu.sync_copy(hist, out_hbm.at[pl.ds(wid * N, N)])


@functools.cache
def _agg_kernel():
    return pl.kernel(
        _agg_body,
        out_type=jax.ShapeDtypeStruct((NC * N, D), jnp.float32),
        mesh=plsc.VectorSubcoreMesh(core_axis_name="c", subcore_axis_name="s"),
        compiler_params=pltpu.CompilerParams(needs_layout_passes=False),
        scratch_types=(
            [pltpu.VMEM_SHARED((N, D), jnp.float32)]
            + [pltpu.VMEM((CH, D), jnp.float32)] * NB
            + [pltpu.VMEM((CH,), jnp.int32)] * NB
            + [pltpu.VMEM((CH,), jnp.int32)] * NB
            + [pltpu.SemaphoreType.DMA] * (3 * NB)
        ),
    )


NRCH = N // CH
RCPT = (NRCH + NS - 1) // NS


def _agg_body(hp_hbm, src_hbm, dst_hbm, out_hbm, acc, *bufs):
    rows = bufs[0:NB]
    sbuf = bufs[NB:2 * NB]
    dbuf = bufs[2 * NB:3 * NB]
    gsem = bufs[3 * NB:4 * NB]
    ssem = bufs[4 * NB:5 * NB]
    isem = bufs[5 * NB:6 * NB]
    c = lax.axis_index("c")
    s = lax.axis_index("s")
    wid = s * NC + c
    ebase = wid * EPW

    def zrow(i, carry):
        for j in range(D // 16):
            rows[0][i, pl.ds(j * 16, 16)] = _zero_vec16()
        return carry

    lax.fori_loop(0, CH, zrow, 0)
    for r in range(RCPT):
        chunk = s + r * NS
        @pl.when(chunk < NRCH)
        def _():
            pltpu.sync_copy(rows[0], acc.at[pl.ds(chunk * CH, CH), :])
    plsc.subcore_barrier()

    def _start_idx(k, b):
        pltpu.async_copy(src_hbm.at[pl.ds(ebase + k * CH, CH)], sbuf[b], isem[b])
        pltpu.async_copy(dst_hbm.at[pl.ds(ebase + k * CH, CH)], dbuf[b], isem[b])

    def _wait_idx(k, b):
        pltpu.make_async_copy(src_hbm.at[pl.ds(ebase + k * CH, CH)], sbuf[b], isem[b]).wait()
        pltpu.make_async_copy(dst_hbm.at[pl.ds(ebase + k * CH, CH)], dbuf[b], isem[b]).wait()

    def _start_gather(b):
        pltpu.async_copy(hp_hbm.at[sbuf[b]], rows[b], gsem[b])

    def _wait_gather(b):
        pltpu.make_async_copy(hp_hbm.at[sbuf[b]], rows[b], gsem[b]).wait()

    def _start_scatter(b):
        pltpu.async_copy(rows[b], acc.at[dbuf[b]], ssem[b], add=True)

    def _wait_scatter(b):
        pltpu.make_async_copy(rows[b], acc.at[dbuf[b]], ssem[b]).wait()

    for b in range(NB):
        _start_idx(b, b)
        _wait_idx(b, b)
        _start_gather(b)

    def body(i, carry):
        for b in range(NB):
            k = i * NB + b
            _wait_gather(b)
            _start_scatter(b)
            @pl.when(k + NB < NCHUNK)
            def _():
                _wait_scatter(b)
                _start_idx(k + NB, b)
                _wait_idx(k + NB, b)
                _start_gather(b)
        return carry

    lax.fori_loop(0, NCHUNK // NB, body, 0)
    _wait_gather(0)
    _start_scatter(0)
    for b in range(NB):
        _wait_scatter(b)

    plsc.subcore_barrier()
    for r in range(RCPT):
        chunk = s + r * NS
        @pl.when(chunk < NRCH)
        def _():
            pltpu.sync_copy(
                acc.at[pl.ds(chunk * CH, CH), :],
                out_hbm.at[pl.ds(c * N + chunk * CH, CH), :],
            )


@functools.cache
def _agg1_kernel():
    return pl.kernel(
        _agg1_body,
        out_type=jax.ShapeDtypeStruct((NW * N,), jnp.float32),
        mesh=plsc.VectorSubcoreMesh(core_axis_name="c", subcore_axis_name="s"),
        compiler_params=pltpu.CompilerParams(needs_layout_passes=False),
        scratch_types=[
            pltpu.VMEM((N,), jnp.float32),
            pltpu.VMEM((N,), jnp.float32),
            pltpu.VMEM((EPW,), jnp.int32),
            pltpu.VMEM((EPW,), jnp.int32),
        ],
    )


def _agg1_body(z_hbm, src_hbm, dst_hbm, out_hbm, zp, acc, sbuf, dbuf):
    c = lax.axis_index("c")
    s = lax.axis_index("s")
    wid = s * NC + c

    pltpu.sync_copy(z_hbm, zp)
    pltpu.sync_copy(src_hbm.at[pl.ds(wid * EPW, EPW)], sbuf)
    pltpu.sync_copy(dst_hbm.at[pl.ds(wid * EPW, EPW)], dbuf)

    def zero_body(i, carry):
        acc[pl.ds(i * 16, 16)] = _zero_vec16()
        return carry

    lax.fori_loop(0, N // 16, zero_body, 0)

    def body(j, carry):
        s16 = sbuf[pl.ds(j * 16, 16)]
        d16 = dbuf[pl.ds(j * 16, 16)]
        vals = plsc.load_gather(zp, [s16])
        plsc.addupdate_scatter(acc, [d16], vals)
        return carry

    lax.fori_loop(0, EPW // 16, body, 0)
    pltpu.sync_copy(acc, out_hbm.at[pl.ds(wid * N, N)])


_RB = 2000


def _prep_body(dp_ref, o_ref):
    deg = jnp.sum(dp_ref[...], axis=0, keepdims=True) + 1.0
    o_ref[...] = lax.rsqrt(deg)


def _prep(degparts):
    return pl.pallas_call(
        _prep_body,
        out_shape=jax.ShapeDtypeStruct((1, N), jnp.float32),
    )(degparts)


def _mm_body(x_ref, w_ref, dinv_ref, o_ref):
    p = jnp.dot(x_ref[...], w_ref[...], preferred_element_type=jnp.float32)
    o_ref[...] = p * dinv_ref[...]


def _mm(x, W, dinv_col):
    kd = W.shape[0]
    od = W.shape[1]
    return pl.pallas_call(
        _mm_body,
        out_shape=jax.ShapeDtypeStruct((N, od), jnp.float32),
        grid=(N // _RB,),
        in_specs=[
            pl.BlockSpec((_RB, kd), lambda i: (i, 0)),
            pl.BlockSpec((kd, od), lambda i: (0, 0)),
            pl.BlockSpec((_RB, 1), lambda i: (i, 0)),
        ],
        out_specs=pl.BlockSpec((_RB, od), lambda i: (i, 0)),
    )(x, W, dinv_col)


def _comb_body(relu, p0_ref, p1_ref, pp_ref, dinv_ref, b_ref, o_ref):
    v = dinv_ref[...] * (p0_ref[...] + p1_ref[...] + pp_ref[...]) + b_ref[...]
    if relu:
        v = jnp.maximum(v, 0.0)
    o_ref[...] = v


def _comb(p0, p1, pp, dinv_col, bias_row, relu):
    return pl.pallas_call(
        functools.partial(_comb_body, relu),
        out_shape=jax.ShapeDtypeStruct((N, D), jnp.float32),
        grid=(N // _RB,),
        in_specs=[
            pl.BlockSpec((_RB, D), lambda i: (i, 0)),
            pl.BlockSpec((_RB, D), lambda i: (i, 0)),
            pl.BlockSpec((_RB, D), lambda i: (i, 0)),
            pl.BlockSpec((_RB, 1), lambda i: (i, 0)),
            pl.BlockSpec((1, D), lambda i: (0, 0)),
        ],
        out_specs=pl.BlockSpec((_RB, D), lambda i: (i, 0)),
    )(p0, p1, pp, dinv_col, bias_row)


def _comb3_body(parts_ref, z_ref, dinv_ref, b_ref, o_ref):
    agg = jnp.sum(parts_ref[...], axis=0, keepdims=True)
    o_ref[...] = dinv_ref[...] * (agg + z_ref[...]) + b_ref[0, 0]


def _comb3(parts3, z_row, dinv_row, b3):
    return pl.pallas_call(
        _comb3_body,
        out_shape=jax.ShapeDtypeStruct((1, N), jnp.float32),
    )(parts3, z_row, dinv_row, b3)


@jax.jit
def kernel(x, edge_index, W1, b1, W2, b2, W3, b3):
    src = edge_index[0].astype(jnp.int32)
    dst = edge_index[1].astype(jnp.int32)

    degparts = _deg_kernel()(dst).reshape(NW, N)
    dinv_row = _prep(degparts)
    dinv_col = dinv_row.reshape(N, 1)

    def layer128(xi, W, b, relu):
        pp = _mm(xi, W, dinv_col)
        parts = _agg_kernel()(pp, src, dst)
        return _comb(parts[:N], parts[N:], pp, dinv_col, b.reshape(1, D), relu)

    h = layer128(x, W1, b1, True)
    h = layer128(h, W2, b2, True)

    z = _mm(h, W3, dinv_col)
    parts3 = _agg1_kernel()(z.reshape(N), src, dst).reshape(NW, N)
    out_row = _comb3(parts3, z.reshape(1, N), dinv_row, b3.reshape(1, 1))
    return out_row.reshape(N, 1)


# speedup vs baseline: 33.4694x; 1.0449x over previous
import functools

import jax
import jax.numpy as jnp
from jax import lax
from jax.experimental import pallas as pl
from jax.experimental.pallas import tpu as pltpu
from jax.experimental.pallas import tpu_sc as plsc

N = 10000
E = 320000
D = 128

NC = 2
NS = 16
NW = NC * NS
EPW = E // NW
CH = 80
NCHUNK = EPW // CH
NB = 4

def _zero_vec16():
    return jnp.zeros((16,), jnp.float32)


@functools.cache
def _deg_kernel():
    return pl.kernel(
        _deg_body,
        out_type=jax.ShapeDtypeStruct((NW * N,), jnp.float32),
        mesh=plsc.VectorSubcoreMesh(core_axis_name="c", subcore_axis_name="s"),
        compiler_params=pltpu.CompilerParams(needs_layout_passes=False),
        scratch_types=[
            pltpu.VMEM((N,), jnp.float32),
            pltpu.VMEM((EPW,), jnp.int32),
        ],
    )


def _deg_body(dst_hbm, out_hbm, hist, dbuf):
    c = lax.axis_index("c")
    s = lax.axis_index("s")
    wid = s * NC + c

    pltpu.sync_copy(dst_hbm.at[pl.ds(wid * EPW, EPW)], dbuf)

    def zero_body(i, carry):
        hist[pl.ds(i * 16, 16)] = _zero_vec16()
        return carry

    lax.fori_loop(0, N // 16, zero_body, 0)

    ones = jnp.ones((16,), jnp.float32)

    def body(j, carry):
        d16 = dbuf[pl.ds(j * 16, 16)]
        plsc.addupdate_scatter(hist, [d16], ones)
        return carry

    lax.fori_loop(0, EPW // 16, body, 0)
    pltpu.sync_copy(hist, out_hbm.at[pl.ds(wid * N, N)])


@functools.cache
def _agg_kernel():
    return pl.kernel(
        _agg_body,
        out_type=jax.ShapeDtypeStruct((NC * N, D), jnp.float32),
        mesh=plsc.VectorSubcoreMesh(core_axis_name="c", subcore_axis_name="s"),
        compiler_params=pltpu.CompilerParams(needs_layout_passes=False),
        scratch_types=(
            [pltpu.VMEM_SHARED((N, D), jnp.float32)]
            + [pltpu.VMEM((CH, D), jnp.float32)] * NB
            + [pltpu.VMEM((CH,), jnp.int32)] * NB
            + [pltpu.VMEM((CH,), jnp.int32)] * NB
            + [pltpu.SemaphoreType.DMA] * (3 * NB)
        ),
    )


NRCH = N // CH
RCPT = (NRCH + NS - 1) // NS


def _agg_body(hp_hbm, src_hbm, dst_hbm, out_hbm, acc, *bufs):
    rows = bufs[0:NB]
    sbuf = bufs[NB:2 * NB]
    dbuf = bufs[2 * NB:3 * NB]
    gsem = bufs[3 * NB:4 * NB]
    ssem = bufs[4 * NB:5 * NB]
    isem = bufs[5 * NB:6 * NB]
    c = lax.axis_index("c")
    s = lax.axis_index("s")
    wid = s * NC + c
    ebase = wid * EPW

    def zrow(i, carry):
        for j in range(D // 16):
            rows[0][i, pl.ds(j * 16, 16)] = _zero_vec16()
        return carry

    lax.fori_loop(0, CH, zrow, 0)
    for r in range(RCPT):
        chunk = s + r * NS
        @pl.when(chunk < NRCH)
        def _():
            pltpu.sync_copy(rows[0], acc.at[pl.ds(chunk * CH, CH), :])
    plsc.subcore_barrier()

    def _start_idx(k, b):
        pltpu.async_copy(src_hbm.at[pl.ds(ebase + k * CH, CH)], sbuf[b], isem[b])
        pltpu.async_copy(dst_hbm.at[pl.ds(ebase + k * CH, CH)], dbuf[b], isem[b])

    def _wait_idx(k, b):
        pltpu.make_async_copy(src_hbm.at[pl.ds(ebase + k * CH, CH)], sbuf[b], isem[b]).wait()
        pltpu.make_async_copy(dst_hbm.at[pl.ds(ebase + k * CH, CH)], dbuf[b], isem[b]).wait()

    def _start_gather(b):
        pltpu.async_copy(hp_hbm.at[sbuf[b]], rows[b], gsem[b])

    def _wait_gather(b):
        pltpu.make_async_copy(hp_hbm.at[sbuf[b]], rows[b], gsem[b]).wait()

    def _start_scatter(b):
        pltpu.async_copy(rows[b], acc.at[dbuf[b]], ssem[b], add=True)

    def _wait_scatter(b):
        pltpu.make_async_copy(rows[b], acc.at[dbuf[b]], ssem[b]).wait()

    for b in range(NB):
        _start_idx(b, b)
        _wait_idx(b, b)
        _start_gather(b)

    def body(i, carry):
        for b in range(NB):
            k = i * NB + b
            _wait_gather(b)
            _start_scatter(b)
            @pl.when(k + NB < NCHUNK)
            def _():
                _wait_scatter(b)
                _start_idx(k + NB, b)
                _wait_idx(k + NB, b)
                _start_gather(b)
        return carry

    lax.fori_loop(0, NCHUNK // NB, body, 0)
    _wait_gather(0)
    _start_scatter(0)
    for b in range(NB):
        _wait_scatter(b)

    plsc.subcore_barrier()
    for r in range(RCPT):
        chunk = s + r * NS
        @pl.when(chunk < NRCH)
        def _():
            pltpu.sync_copy(
                acc.at[pl.ds(chunk * CH, CH), :],
                out_hbm.at[pl.ds(c * N + chunk * CH, CH), :],
            )


@functools.cache
def _agg1_kernel():
    return pl.kernel(
        _agg1_body,
        out_type=jax.ShapeDtypeStruct((NW * N,), jnp.float32),
        mesh=plsc.VectorSubcoreMesh(core_axis_name="c", subcore_axis_name="s"),
        compiler_params=pltpu.CompilerParams(needs_layout_passes=False),
        scratch_types=[
            pltpu.VMEM((N,), jnp.float32),
            pltpu.VMEM((N,), jnp.float32),
            pltpu.VMEM((EPW,), jnp.int32),
            pltpu.VMEM((EPW,), jnp.int32),
        ],
    )


def _agg1_body(z_hbm, src_hbm, dst_hbm, out_hbm, zp, acc, sbuf, dbuf):
    c = lax.axis_index("c")
    s = lax.axis_index("s")
    wid = s * NC + c

    pltpu.sync_copy(z_hbm, zp)
    pltpu.sync_copy(src_hbm.at[pl.ds(wid * EPW, EPW)], sbuf)
    pltpu.sync_copy(dst_hbm.at[pl.ds(wid * EPW, EPW)], dbuf)

    def zero_body(i, carry):
        acc[pl.ds(i * 16, 16)] = _zero_vec16()
        return carry

    lax.fori_loop(0, N // 16, zero_body, 0)

    def body(j, carry):
        s16 = sbuf[pl.ds(j * 16, 16)]
        d16 = dbuf[pl.ds(j * 16, 16)]
        vals = plsc.load_gather(zp, [s16])
        plsc.addupdate_scatter(acc, [d16], vals)
        return carry

    lax.fori_loop(0, EPW // 16, body, 0)
    pltpu.sync_copy(acc, out_hbm.at[pl.ds(wid * N, N)])


_RB = 2000


def _prep_body(dp_ref, or_ref, oc_ref):
    deg = jnp.sum(dp_ref[...], axis=0, keepdims=True) + 1.0
    dinv = lax.rsqrt(deg)
    or_ref[...] = dinv
    oc_ref[...] = dinv.T


def _prep(degparts):
    return pl.pallas_call(
        _prep_body,
        out_shape=[
            jax.ShapeDtypeStruct((1, N), jnp.float32),
            jax.ShapeDtypeStruct((N, 1), jnp.float32),
        ],
    )(degparts)


def _mm_body(x_ref, w_ref, dinv_ref, o_ref):
    p = jnp.dot(x_ref[...], w_ref[...], preferred_element_type=jnp.float32)
    o_ref[...] = p * dinv_ref[...]


def _mm(x, W, dinv_col):
    kd = W.shape[0]
    od = W.shape[1]
    return pl.pallas_call(
        _mm_body,
        out_shape=jax.ShapeDtypeStruct((N, od), jnp.float32),
        grid=(N // _RB,),
        in_specs=[
            pl.BlockSpec((_RB, kd), lambda i: (i, 0)),
            pl.BlockSpec((kd, od), lambda i: (0, 0)),
            pl.BlockSpec((_RB, 1), lambda i: (i, 0)),
        ],
        out_specs=pl.BlockSpec((_RB, od), lambda i: (i, 0)),
    )(x, W, dinv_col)


def _combmm_body(p0_ref, p1_ref, pp_ref, dinv_ref, b_ref, w_ref, o_ref):
    t = dinv_ref[...] * (p0_ref[...] + p1_ref[...] + pp_ref[...]) + b_ref[...]
    t = jnp.maximum(t, 0.0)
    p = jnp.dot(t, w_ref[...], preferred_element_type=jnp.float32)
    o_ref[...] = p * dinv_ref[...]


def _combmm(p0, p1, pp, dinv_col, bias_row, Wn):
    od = Wn.shape[1]
    return pl.pallas_call(
        _combmm_body,
        out_shape=jax.ShapeDtypeStruct((N, od), jnp.float32),
        grid=(N // _RB,),
        in_specs=[
            pl.BlockSpec((_RB, D), lambda i: (i, 0)),
            pl.BlockSpec((_RB, D), lambda i: (i, 0)),
            pl.BlockSpec((_RB, D), lambda i: (i, 0)),
            pl.BlockSpec((_RB, 1), lambda i: (i, 0)),
            pl.BlockSpec((1, D), lambda i: (0, 0)),
            pl.BlockSpec((D, od), lambda i: (0, 0)),
        ],
        out_specs=pl.BlockSpec((_RB, od), lambda i: (i, 0)),
    )(p0, p1, pp, dinv_col, bias_row, Wn)


def _comb3_body(parts_ref, z_ref, dinv_ref, b_ref, o_ref):
    agg = jnp.sum(parts_ref[...], axis=0, keepdims=True)
    o_ref[...] = dinv_ref[...] * (agg + z_ref[...]) + b_ref[0, 0]


def _comb3(parts3, z_row, dinv_row, b3):
    return pl.pallas_call(
        _comb3_body,
        out_shape=jax.ShapeDtypeStruct((1, N), jnp.float32),
    )(parts3, z_row, dinv_row, b3)


@jax.jit
def kernel(x, edge_index, W1, b1, W2, b2, W3, b3):
    src = edge_index[0].astype(jnp.int32)
    dst = edge_index[1].astype(jnp.int32)

    degparts = _deg_kernel()(dst).reshape(NW, N)
    dinv_row, dinv_col = _prep(degparts)

    pp1 = _mm(x, W1, dinv_col)
    parts = _agg_kernel()(pp1, src, dst)
    pp2 = _combmm(parts[:N], parts[N:], pp1, dinv_col, b1.reshape(1, D), W2)
    parts = _agg_kernel()(pp2, src, dst)
    z = _combmm(parts[:N], parts[N:], pp2, dinv_col, b2.reshape(1, D), W3)

    parts3 = _agg1_kernel()(z.reshape(N), src, dst).reshape(NW, N)
    out_row = _comb3(parts3, z.reshape(1, N), dinv_row, b3.reshape(1, 1))
    return out_row.reshape(N, 1)
